# baseline (speedup 1.0000x reference)
import jax
import jax.numpy as jnp
from jax import lax
from jax.experimental import pallas as pl
from jax.experimental.pallas import tpu as pltpu
from jax.experimental.pallas import tpu_sc as plsc

DM = 1024
SL = 8192

_info = plsc.get_sparse_core_info()
_NC, _NS, _L = _info.num_cores, _info.num_subcores, _info.num_lanes
_NW = _NC * _NS
_ROWS_PER_W = SL // _NW
_CH = 32
_NCHUNK = _ROWS_PER_W // _CH
_CHW = _CH * DM
_NVEC = _CHW // _L


def _tc_body(x_ref, t_ref, o_ref):
    o_ref[...] = x_ref[...] + t_ref[...]


def _sc_body(x_hbm, t_hbm, o_hbm, xbuf, tbuf, obuf):
    wid = lax.axis_index("s") * _NC + lax.axis_index("c")
    row0 = wid * _ROWS_PER_W

    def do_chunk(ci, _):
        r = (row0 + ci * _CH) * DM
        pltpu.sync_copy(t_hbm.at[pl.ds(r, _CHW)], tbuf)
        pltpu.sync_copy(x_hbm.at[pl.ds(3 * SL * DM + r, _CHW)], xbuf)

        @plsc.parallel_loop(0, _NVEC, unroll=8)
        def _add_vec(i):
            s = pl.ds(i * _L, _L)
            obuf[s] = xbuf[s] + tbuf[s]

        pltpu.sync_copy(obuf, o_hbm.at[pl.ds(r, _CHW)])
        return 0

    lax.fori_loop(0, _NCHUNK, do_chunk, 0)


def kernel(x, embedding_table):
    B, S, D = x.shape
    BS = 2048
    xf = x.reshape(-1)
    tf = embedding_table.reshape(-1)

    tc_out = pl.pallas_call(
        _tc_body,
        grid=(S // BS, B - 1),
        in_specs=[
            pl.BlockSpec((1, BS, D), lambda s, b: (b, s, 0)),
            pl.BlockSpec((BS, D), lambda s, b: (s, 0)),
        ],
        out_specs=pl.BlockSpec((1, BS, D), lambda s, b: (b, s, 0)),
        out_shape=jax.ShapeDtypeStruct((B - 1, S, D), x.dtype),
    )(x, embedding_table)

    sc_run = pl.kernel(
        _sc_body,
        out_type=jax.ShapeDtypeStruct((S * D,), jnp.float32),
        mesh=plsc.VectorSubcoreMesh(core_axis_name="c", subcore_axis_name="s"),
        scratch_types=[
            pltpu.VMEM((_CHW,), jnp.float32),
            pltpu.VMEM((_CHW,), jnp.float32),
            pltpu.VMEM((_CHW,), jnp.float32),
        ],
    )
    sc_out = sc_run(xf, tf).reshape(1, S, D)

    return jnp.concatenate([tc_out, sc_out], axis=0)


# speedup vs baseline: 3.9134x; 3.9134x over previous
import jax
import jax.numpy as jnp
from jax.experimental import pallas as pl
from jax.experimental.pallas import tpu as pltpu


def _add_body(x_ref, t_ref, o_ref):
    o_ref[...] = x_ref[...] + t_ref[...]


def kernel(x, embedding_table):
    B, S, D = x.shape
    BS = 2048
    grid = (S // BS, B)
    return pl.pallas_call(
        _add_body,
        grid=grid,
        in_specs=[
            pl.BlockSpec((1, BS, D), lambda s, b: (b, s, 0)),
            pl.BlockSpec((BS, D), lambda s, b: (s, 0)),
        ],
        out_specs=pl.BlockSpec((1, BS, D), lambda s, b: (b, s, 0)),
        out_shape=jax.ShapeDtypeStruct(x.shape, x.dtype),
        compiler_params=pltpu.CompilerParams(
            dimension_semantics=("parallel", "parallel"),
        ),
    )(x, embedding_table)
